# baseline (speedup 1.0000x reference)
import math

import jax
import jax.numpy as jnp
from jax import lax
from jax.experimental import pallas as pl
from jax.experimental.pallas import tpu as pltpu
from jax.experimental.pallas import tpu_sc as plsc

B, S, D = 1, 2048, 768
RANK = 64
N_COMPRESS = 8
N_KNOWLEDGE = 100000
KNOWLEDGE_K = 8

TT = 512
TK = 2048
NKT = -(-N_KNOWLEDGE // TK)
MINF = -3.0e38
SCALE = 1.0 / math.sqrt(RANK)
_LANES = 16
WREP = KNOWLEDGE_K * _LANES


def _phase1_body(x_ref, wrt_ref, c2_ref, k_ref, idx_out, w_out,
                 q_scr, tops_scr, topi_scr, s_scr):
    kstep = pl.program_id(1)

    @pl.when(kstep == 0)
    def _init():
        X = x_ref[...]
        r = jnp.dot(X, wrt_ref[...],
                    preferred_element_type=jnp.float32)
        r = r - jnp.max(r, axis=1, keepdims=True)
        e = jnp.exp(r)
        w8 = e / jnp.sum(e, axis=1, keepdims=True)
        q = jnp.zeros((TT, RANK), jnp.float32)
        for n in range(N_COMPRESS):
            pn = jnp.dot(X, c2_ref[:, n * RANK:(n + 1) * RANK],
                         preferred_element_type=jnp.float32)
            q = q + pn * w8[:, n:n + 1]
        q_scr[...] = q
        tops_scr[...] = jnp.full((TT, KNOWLEDGE_K), MINF, jnp.float32)
        topi_scr[...] = jnp.zeros((TT, KNOWLEDGE_K), jnp.int32)

    base = kstep * TK
    scores = lax.dot_general(q_scr[...], k_ref[...],
                             (((1,), (1,)), ((), ())),
                             preferred_element_type=jnp.float32) * SCALE
    col = lax.broadcasted_iota(jnp.int32, (TT, TK), 1)
    valid = (col + base) < N_KNOWLEDGE
    scores = jnp.where(valid, scores, MINF)

    ts0 = tops_scr[...]
    ti0 = topi_scr[...]
    thresh = ts0[:, KNOWLEDGE_K - 1:KNOWLEDGE_K]
    cnt = jnp.sum((scores > thresh).astype(jnp.int32), axis=1)
    needed = jnp.max(jnp.minimum(cnt, KNOWLEDGE_K))
    s_scr[...] = scores

    js = lax.broadcasted_iota(jnp.int32, (TT, KNOWLEDGE_K), 1)

    def round_body(_, carry):
        ts, ti = carry
        s = s_scr[...]
        m = jnp.max(s, axis=1, keepdims=True)
        am = jnp.min(jnp.where(s == m, col, jnp.int32(1 << 30)),
                     axis=1, keepdims=True)
        s_scr[...] = jnp.where(col == am, MINF, s)
        gam = am + base
        pos = jnp.sum((ts >= m).astype(jnp.int32), axis=1, keepdims=True)
        sh_ts = jnp.concatenate([ts[:, :1], ts[:, :KNOWLEDGE_K - 1]], axis=1)
        sh_ti = jnp.concatenate([ti[:, :1], ti[:, :KNOWLEDGE_K - 1]], axis=1)
        nts = jnp.where(js < pos, ts, jnp.where(js == pos, m, sh_ts))
        nti = jnp.where(js < pos, ti, jnp.where(js == pos, gam, sh_ti))
        return nts, nti

    ts_f, ti_f = lax.fori_loop(0, needed, round_body, (ts0, ti0))
    tops_scr[...] = ts_f
    topi_scr[...] = ti_f

    @pl.when(kstep == NKT - 1)
    def _emit():
        ts = tops_scr[...]
        e = jnp.exp(ts - ts[:, :1])
        w = e / jnp.sum(e, axis=1, keepdims=True)
        w_out[...] = jnp.repeat(w, _LANES, axis=1)
        idx_out[...] = topi_scr[...]


def _phase1(x2d, wrt, c2, knowledge_K):
    ntok = x2d.shape[0]
    return pl.pallas_call(
        _phase1_body,
        grid=(ntok // TT, NKT),
        in_specs=[
            pl.BlockSpec((TT, D), lambda t, k: (t, 0)),
            pl.BlockSpec((D, N_COMPRESS), lambda t, k: (0, 0)),
            pl.BlockSpec((D, N_COMPRESS * RANK), lambda t, k: (0, 0)),
            pl.BlockSpec((TK, RANK), lambda t, k: (k, 0)),
        ],
        out_specs=[
            pl.BlockSpec((TT, KNOWLEDGE_K), lambda t, k: (t, 0)),
            pl.BlockSpec((TT, WREP), lambda t, k: (t, 0)),
        ],
        out_shape=[
            jax.ShapeDtypeStruct((ntok, KNOWLEDGE_K), jnp.int32),
            jax.ShapeDtypeStruct((ntok, WREP), jnp.float32),
        ],
        scratch_shapes=[
            pltpu.VMEM((TT, RANK), jnp.float32),
            pltpu.VMEM((TT, KNOWLEDGE_K), jnp.float32),
            pltpu.VMEM((TT, KNOWLEDGE_K), jnp.int32),
            pltpu.VMEM((TT, TK), jnp.float32),
        ],
    )(x2d, wrt, c2, knowledge_K)


_NC = 2
_NS = 16
_NW = _NC * _NS
_TCH = 8
_DCH = D // _LANES


def _make_sc_body(ntok):
    tpw = ntok // _NW
    nchunk = tpw // _TCH

    def _sc_body(v_hbm, idx_hbm, w_hbm, out_hbm, idx_v, rows_v, w_v, out_v,
                 sem):
        wid = lax.axis_index("s") * _NC + lax.axis_index("c")

        def chunk(ch, _):
            tok0 = wid * tpw + ch * _TCH
            row0 = tok0 * KNOWLEDGE_K
            pltpu.sync_copy(idx_hbm.at[pl.ds(row0, _TCH * KNOWLEDGE_K)], idx_v)
            pltpu.async_copy(v_hbm.at[idx_v], rows_v, sem).wait()
            pltpu.sync_copy(w_hbm.at[pl.ds(tok0, _TCH)], w_v)

            def tok(tt, _):
                def lane_chunk(cc, _):
                    acc = rows_v[tt * KNOWLEDGE_K, pl.ds(cc * _LANES, _LANES)] \
                        * w_v[tt, 0, :]
                    for j in range(1, KNOWLEDGE_K):
                        acc = acc + rows_v[tt * KNOWLEDGE_K + j,
                                           pl.ds(cc * _LANES, _LANES)] \
                            * w_v[tt, j, :]
                    out_v[tt, pl.ds(cc * _LANES, _LANES)] = acc
                    return 0

                lax.fori_loop(0, _DCH, lane_chunk, 0)
                return 0

            lax.fori_loop(0, _TCH, tok, 0)
            pltpu.sync_copy(out_v, out_hbm.at[pl.ds(tok0, _TCH)])
            return 0

        lax.fori_loop(0, nchunk, chunk, 0)

    return _sc_body


def _phase2(knowledge_V, idx_flat, w_rep):
    ntok = w_rep.shape[0]
    mesh = plsc.VectorSubcoreMesh(core_axis_name="c", subcore_axis_name="s")
    f = pl.kernel(
        _make_sc_body(ntok), mesh=mesh,
        out_type=jax.ShapeDtypeStruct((ntok, D), jnp.float32),
        scratch_types=[
            pltpu.VMEM((_TCH * KNOWLEDGE_K,), jnp.int32),
            pltpu.VMEM((_TCH * KNOWLEDGE_K, D), jnp.float32),
            pltpu.VMEM((_TCH, KNOWLEDGE_K, _LANES), jnp.float32),
            pltpu.VMEM((_TCH, D), jnp.float32),
            pltpu.SemaphoreType.DMA,
        ],
    )
    return f(knowledge_V, idx_flat, w_rep)


def kernel(x, W_router, compress_neurons, knowledge_K, knowledge_V):
    x2d = x.reshape(S, D)
    wrt = W_router.T
    c2 = jnp.transpose(compress_neurons, (1, 0, 2)).reshape(D, N_COMPRESS * RANK)

    halves = []
    h = S // 2
    for i in range(2):
        xh = x2d[i * h:(i + 1) * h]
        topk_idx, w128 = _phase1(xh, wrt, c2, knowledge_K)
        idx_flat = topk_idx.reshape(h * KNOWLEDGE_K)
        w_rep = w128.reshape(h, KNOWLEDGE_K, _LANES)
        halves.append(_phase2(knowledge_V, idx_flat, w_rep))
    out = jnp.concatenate(halves, axis=0)
    return out.reshape(B, S, D)


# speedup vs baseline: 1.0149x; 1.0149x over previous
import math

import jax
import jax.numpy as jnp
from jax import lax
from jax.experimental import pallas as pl
from jax.experimental.pallas import tpu as pltpu
from jax.experimental.pallas import tpu_sc as plsc

B, S, D = 1, 2048, 768
RANK = 64
N_COMPRESS = 8
N_KNOWLEDGE = 100000
KNOWLEDGE_K = 8

TT = 256
TK = 2048
NKT = -(-N_KNOWLEDGE // TK)
MINF = -3.0e38
SCALE = 1.0 / math.sqrt(RANK)
_LANES = 16
WREP = KNOWLEDGE_K * _LANES


def _phase1_body(x_ref, wrt_ref, c2_ref, k_ref, idx_out, w_out,
                 q_scr, tops_scr, topi_scr, s_scr):
    kstep = pl.program_id(1)

    @pl.when(kstep == 0)
    def _init():
        X = x_ref[...]
        r = jnp.dot(X, wrt_ref[...],
                    preferred_element_type=jnp.float32)
        r = r - jnp.max(r, axis=1, keepdims=True)
        e = jnp.exp(r)
        w8 = e / jnp.sum(e, axis=1, keepdims=True)
        q = jnp.zeros((TT, RANK), jnp.float32)
        for n in range(N_COMPRESS):
            pn = jnp.dot(X, c2_ref[:, n * RANK:(n + 1) * RANK],
                         preferred_element_type=jnp.float32)
            q = q + pn * w8[:, n:n + 1]
        q_scr[...] = q
        tops_scr[...] = jnp.full((TT, KNOWLEDGE_K), MINF, jnp.float32)
        topi_scr[...] = jnp.zeros((TT, KNOWLEDGE_K), jnp.int32)

    base = kstep * TK
    scores = lax.dot_general(q_scr[...], k_ref[...],
                             (((1,), (1,)), ((), ())),
                             preferred_element_type=jnp.float32) * SCALE
    col = lax.broadcasted_iota(jnp.int32, (TT, TK), 1)
    valid = (col + base) < N_KNOWLEDGE
    scores = jnp.where(valid, scores, MINF)

    ts0 = tops_scr[...]
    ti0 = topi_scr[...]
    thresh = ts0[:, KNOWLEDGE_K - 1:KNOWLEDGE_K]
    cnt = jnp.sum((scores > thresh).astype(jnp.int32), axis=1)
    needed = jnp.max(jnp.minimum(cnt, KNOWLEDGE_K))
    s_scr[...] = scores

    js = lax.broadcasted_iota(jnp.int32, (TT, KNOWLEDGE_K), 1)

    def round_body(_, carry):
        ts, ti = carry
        s = s_scr[...]
        m = jnp.max(s, axis=1, keepdims=True)
        am = jnp.min(jnp.where(s == m, col, jnp.int32(1 << 30)),
                     axis=1, keepdims=True)
        s_scr[...] = jnp.where(col == am, MINF, s)
        gam = am + base
        pos = jnp.sum((ts >= m).astype(jnp.int32), axis=1, keepdims=True)
        sh_ts = jnp.concatenate([ts[:, :1], ts[:, :KNOWLEDGE_K - 1]], axis=1)
        sh_ti = jnp.concatenate([ti[:, :1], ti[:, :KNOWLEDGE_K - 1]], axis=1)
        nts = jnp.where(js < pos, ts, jnp.where(js == pos, m, sh_ts))
        nti = jnp.where(js < pos, ti, jnp.where(js == pos, gam, sh_ti))
        return nts, nti

    ts_f, ti_f = lax.fori_loop(0, needed, round_body, (ts0, ti0))
    tops_scr[...] = ts_f
    topi_scr[...] = ti_f

    @pl.when(kstep == NKT - 1)
    def _emit():
        ts = tops_scr[...]
        e = jnp.exp(ts - ts[:, :1])
        w = e / jnp.sum(e, axis=1, keepdims=True)
        w_out[...] = jnp.repeat(w, _LANES, axis=1)
        idx_out[...] = topi_scr[...]


def _phase1(x2d, wrt, c2, knowledge_K):
    ntok = x2d.shape[0]
    return pl.pallas_call(
        _phase1_body,
        grid=(ntok // TT, NKT),
        in_specs=[
            pl.BlockSpec((TT, D), lambda t, k: (t, 0)),
            pl.BlockSpec((D, N_COMPRESS), lambda t, k: (0, 0)),
            pl.BlockSpec((D, N_COMPRESS * RANK), lambda t, k: (0, 0)),
            pl.BlockSpec((TK, RANK), lambda t, k: (k, 0)),
        ],
        out_specs=[
            pl.BlockSpec((TT, KNOWLEDGE_K), lambda t, k: (t, 0)),
            pl.BlockSpec((TT, WREP), lambda t, k: (t, 0)),
        ],
        out_shape=[
            jax.ShapeDtypeStruct((ntok, KNOWLEDGE_K), jnp.int32),
            jax.ShapeDtypeStruct((ntok, WREP), jnp.float32),
        ],
        scratch_shapes=[
            pltpu.VMEM((TT, RANK), jnp.float32),
            pltpu.VMEM((TT, KNOWLEDGE_K), jnp.float32),
            pltpu.VMEM((TT, KNOWLEDGE_K), jnp.int32),
            pltpu.VMEM((TT, TK), jnp.float32),
        ],
    )(x2d, wrt, c2, knowledge_K)


_NC = 2
_NS = 16
_NW = _NC * _NS
_TCH = 8
_DCH = D // _LANES


def _make_sc_body(ntok):
    tpw = ntok // _NW
    nchunk = tpw // _TCH

    def _sc_body(v_hbm, idx_hbm, w_hbm, out_hbm, idx_v, rows_v, w_v, out_v,
                 sem):
        wid = lax.axis_index("s") * _NC + lax.axis_index("c")

        def chunk(ch, _):
            tok0 = wid * tpw + ch * _TCH
            row0 = tok0 * KNOWLEDGE_K
            pltpu.sync_copy(idx_hbm.at[pl.ds(row0, _TCH * KNOWLEDGE_K)], idx_v)
            pltpu.async_copy(v_hbm.at[idx_v], rows_v, sem).wait()
            pltpu.sync_copy(w_hbm.at[pl.ds(tok0, _TCH)], w_v)

            def tok(tt, _):
                def lane_chunk(cc, _):
                    acc = rows_v[tt * KNOWLEDGE_K, pl.ds(cc * _LANES, _LANES)] \
                        * w_v[tt, 0, :]
                    for j in range(1, KNOWLEDGE_K):
                        acc = acc + rows_v[tt * KNOWLEDGE_K + j,
                                           pl.ds(cc * _LANES, _LANES)] \
                            * w_v[tt, j, :]
                    out_v[tt, pl.ds(cc * _LANES, _LANES)] = acc
                    return 0

                lax.fori_loop(0, _DCH, lane_chunk, 0)
                return 0

            lax.fori_loop(0, _TCH, tok, 0)
            pltpu.sync_copy(out_v, out_hbm.at[pl.ds(tok0, _TCH)])
            return 0

        lax.fori_loop(0, nchunk, chunk, 0)

    return _sc_body


def _phase2(knowledge_V, idx_flat, w_rep):
    ntok = w_rep.shape[0]
    mesh = plsc.VectorSubcoreMesh(core_axis_name="c", subcore_axis_name="s")
    f = pl.kernel(
        _make_sc_body(ntok), mesh=mesh,
        out_type=jax.ShapeDtypeStruct((ntok, D), jnp.float32),
        scratch_types=[
            pltpu.VMEM((_TCH * KNOWLEDGE_K,), jnp.int32),
            pltpu.VMEM((_TCH * KNOWLEDGE_K, D), jnp.float32),
            pltpu.VMEM((_TCH, KNOWLEDGE_K, _LANES), jnp.float32),
            pltpu.VMEM((_TCH, D), jnp.float32),
            pltpu.SemaphoreType.DMA,
        ],
    )
    return f(knowledge_V, idx_flat, w_rep)


def kernel(x, W_router, compress_neurons, knowledge_K, knowledge_V):
    x2d = x.reshape(S, D)
    wrt = W_router.T
    c2 = jnp.transpose(compress_neurons, (1, 0, 2)).reshape(D, N_COMPRESS * RANK)

    halves = []
    h = S // 2
    for i in range(2):
        xh = x2d[i * h:(i + 1) * h]
        topk_idx, w128 = _phase1(xh, wrt, c2, knowledge_K)
        idx_flat = topk_idx.reshape(h * KNOWLEDGE_K)
        w_rep = w128.reshape(h, KNOWLEDGE_K, _LANES)
        halves.append(_phase2(knowledge_V, idx_flat, w_rep))
    out = jnp.concatenate(halves, axis=0)
    return out.reshape(B, S, D)


# speedup vs baseline: 1.0167x; 1.0018x over previous
import math

import jax
import jax.numpy as jnp
from jax import lax
from jax.experimental import pallas as pl
from jax.experimental.pallas import tpu as pltpu
from jax.experimental.pallas import tpu_sc as plsc

B, S, D = 1, 2048, 768
RANK = 64
N_COMPRESS = 8
N_KNOWLEDGE = 100000
KNOWLEDGE_K = 8

TT = 256
TK = 2048
NKT = -(-N_KNOWLEDGE // TK)
MINF = -3.0e38
SCALE = 1.0 / math.sqrt(RANK)
_LANES = 16
WREP = KNOWLEDGE_K * _LANES


def _phase1_body(x_ref, wrt_ref, c2_ref, k_ref, idx_out, w_out,
                 q_scr, tops_scr, topi_scr, s_scr):
    kstep = pl.program_id(1)

    @pl.when(kstep == 0)
    def _init():
        X = x_ref[...]
        r = jnp.dot(X, wrt_ref[...],
                    preferred_element_type=jnp.float32)
        r = r - jnp.max(r, axis=1, keepdims=True)
        e = jnp.exp(r)
        w8 = e / jnp.sum(e, axis=1, keepdims=True)
        q = jnp.zeros((TT, RANK), jnp.float32)
        for n in range(N_COMPRESS):
            pn = jnp.dot(X, c2_ref[:, n * RANK:(n + 1) * RANK],
                         preferred_element_type=jnp.float32)
            q = q + pn * w8[:, n:n + 1]
        q_scr[...] = q
        tops_scr[...] = jnp.full((TT, KNOWLEDGE_K), MINF, jnp.float32)
        topi_scr[...] = jnp.zeros((TT, KNOWLEDGE_K), jnp.int32)

    base = kstep * TK
    scores = lax.dot_general(q_scr[...], k_ref[...],
                             (((1,), (1,)), ((), ())),
                             preferred_element_type=jnp.float32) * SCALE
    col = lax.broadcasted_iota(jnp.int32, (TT, TK), 1)
    valid = (col + base) < N_KNOWLEDGE
    scores = jnp.where(valid, scores, MINF)

    ts0 = tops_scr[...]
    ti0 = topi_scr[...]
    thresh = ts0[:, KNOWLEDGE_K - 1:KNOWLEDGE_K]
    cnt = jnp.sum((scores > thresh).astype(jnp.int32), axis=1)
    needed = jnp.max(jnp.minimum(cnt, KNOWLEDGE_K))
    s_scr[...] = scores

    js = lax.broadcasted_iota(jnp.int32, (TT, KNOWLEDGE_K), 1)

    def round_body(_, carry):
        ts, ti = carry
        s = s_scr[...]
        m = jnp.max(s, axis=1, keepdims=True)
        am = jnp.min(jnp.where(s == m, col, jnp.int32(1 << 30)),
                     axis=1, keepdims=True)
        s_scr[...] = jnp.where(col == am, MINF, s)
        gam = am + base
        pos = jnp.sum((ts >= m).astype(jnp.int32), axis=1, keepdims=True)
        sh_ts = jnp.concatenate([ts[:, :1], ts[:, :KNOWLEDGE_K - 1]], axis=1)
        sh_ti = jnp.concatenate([ti[:, :1], ti[:, :KNOWLEDGE_K - 1]], axis=1)
        nts = jnp.where(js < pos, ts, jnp.where(js == pos, m, sh_ts))
        nti = jnp.where(js < pos, ti, jnp.where(js == pos, gam, sh_ti))
        return nts, nti

    ts_f, ti_f = lax.fori_loop(0, needed, round_body, (ts0, ti0))
    tops_scr[...] = ts_f
    topi_scr[...] = ti_f

    @pl.when(kstep == NKT - 1)
    def _emit():
        ts = tops_scr[...]
        e = jnp.exp(ts - ts[:, :1])
        w = e / jnp.sum(e, axis=1, keepdims=True)
        w_out[...] = jnp.repeat(w, _LANES, axis=1)
        idx_out[...] = topi_scr[...]


def _phase1(x2d, wrt, c2, knowledge_K):
    ntok = x2d.shape[0]
    return pl.pallas_call(
        _phase1_body,
        grid=(ntok // TT, NKT),
        in_specs=[
            pl.BlockSpec((TT, D), lambda t, k: (t, 0)),
            pl.BlockSpec((D, N_COMPRESS), lambda t, k: (0, 0)),
            pl.BlockSpec((D, N_COMPRESS * RANK), lambda t, k: (0, 0)),
            pl.BlockSpec((TK, RANK), lambda t, k: (k, 0)),
        ],
        out_specs=[
            pl.BlockSpec((TT, KNOWLEDGE_K), lambda t, k: (t, 0)),
            pl.BlockSpec((TT, WREP), lambda t, k: (t, 0)),
        ],
        out_shape=[
            jax.ShapeDtypeStruct((ntok, KNOWLEDGE_K), jnp.int32),
            jax.ShapeDtypeStruct((ntok, WREP), jnp.float32),
        ],
        scratch_shapes=[
            pltpu.VMEM((TT, RANK), jnp.float32),
            pltpu.VMEM((TT, KNOWLEDGE_K), jnp.float32),
            pltpu.VMEM((TT, KNOWLEDGE_K), jnp.int32),
            pltpu.VMEM((TT, TK), jnp.float32),
        ],
    )(x2d, wrt, c2, knowledge_K)


_NC = 2
_NS = 16
_NW = _NC * _NS
_TCH = 16
_DCH = D // _LANES


def _make_sc_body(ntok):
    tpw = ntok // _NW
    nchunk = tpw // _TCH

    def _sc_body(v_hbm, idx_hbm, w_hbm, out_hbm, idx_v, rows_v, w_v, out_v,
                 sem):
        wid = lax.axis_index("s") * _NC + lax.axis_index("c")

        def chunk(ch, _):
            tok0 = wid * tpw + ch * _TCH
            row0 = tok0 * KNOWLEDGE_K
            pltpu.sync_copy(idx_hbm.at[pl.ds(row0, _TCH * KNOWLEDGE_K)], idx_v)
            pltpu.async_copy(v_hbm.at[idx_v], rows_v, sem).wait()
            pltpu.sync_copy(w_hbm.at[pl.ds(tok0, _TCH)], w_v)

            def tok(tt, _):
                def lane_chunk(cc, _):
                    acc = rows_v[tt * KNOWLEDGE_K, pl.ds(cc * _LANES, _LANES)] \
                        * w_v[tt, 0, :]
                    for j in range(1, KNOWLEDGE_K):
                        acc = acc + rows_v[tt * KNOWLEDGE_K + j,
                                           pl.ds(cc * _LANES, _LANES)] \
                            * w_v[tt, j, :]
                    out_v[tt, pl.ds(cc * _LANES, _LANES)] = acc
                    return 0

                lax.fori_loop(0, _DCH, lane_chunk, 0)
                return 0

            lax.fori_loop(0, _TCH, tok, 0)
            pltpu.sync_copy(out_v, out_hbm.at[pl.ds(tok0, _TCH)])
            return 0

        lax.fori_loop(0, nchunk, chunk, 0)

    return _sc_body


def _phase2(knowledge_V, idx_flat, w_rep):
    ntok = w_rep.shape[0]
    mesh = plsc.VectorSubcoreMesh(core_axis_name="c", subcore_axis_name="s")
    f = pl.kernel(
        _make_sc_body(ntok), mesh=mesh,
        out_type=jax.ShapeDtypeStruct((ntok, D), jnp.float32),
        scratch_types=[
            pltpu.VMEM((_TCH * KNOWLEDGE_K,), jnp.int32),
            pltpu.VMEM((_TCH * KNOWLEDGE_K, D), jnp.float32),
            pltpu.VMEM((_TCH, KNOWLEDGE_K, _LANES), jnp.float32),
            pltpu.VMEM((_TCH, D), jnp.float32),
            pltpu.SemaphoreType.DMA,
        ],
    )
    return f(knowledge_V, idx_flat, w_rep)


def kernel(x, W_router, compress_neurons, knowledge_K, knowledge_V):
    x2d = x.reshape(S, D)
    wrt = W_router.T
    c2 = jnp.transpose(compress_neurons, (1, 0, 2)).reshape(D, N_COMPRESS * RANK)

    halves = []
    h = S // 2
    for i in range(2):
        xh = x2d[i * h:(i + 1) * h]
        topk_idx, w128 = _phase1(xh, wrt, c2, knowledge_K)
        idx_flat = topk_idx.reshape(h * KNOWLEDGE_K)
        w_rep = w128.reshape(h, KNOWLEDGE_K, _LANES)
        halves.append(_phase2(knowledge_V, idx_flat, w_rep))
    out = jnp.concatenate(halves, axis=0)
    return out.reshape(B, S, D)
